# baseline (speedup 1.0000x reference)
import functools

import jax
import jax.numpy as jnp
from jax import lax
from jax.experimental import pallas as pl
from jax.experimental.pallas import tpu as pltpu

N = 100000
F = 128
G = 1000
B = 5000
NT = N // B
W = 128
NW = (G + W - 1) // W
GP = NW * W
EPS = 1e-5


def _fused_body(x_ref, seg_ref, w1_ref, b1_ref, w2_ref, b2_ref, g_ref,
                be_ref, out_ref, accp_ref, accc_ref, q_ref):
    i = pl.program_id(0)

    @pl.when(i == 0)
    def _init():
        accp_ref[...] = jnp.zeros_like(accp_ref)
        accc_ref[...] = jnp.zeros_like(accc_ref)
        q_ref[...] = jnp.zeros_like(q_ref)

    x = x_ref[...]
    l1 = lax.dot_general(x, w1_ref[...], (((1,), (1,)), ((), ())),
                         preferred_element_type=jnp.float32) + b1_ref[...]
    l2 = lax.dot_general(x, w2_ref[...], (((1,), (1,)), ((), ())),
                         preferred_element_type=jnp.float32) + b2_ref[...]
    h = l1 * l2

    q_ref[...] += jnp.sum(h * h, axis=0, keepdims=True)

    seg = seg_ref[0]
    lo = jnp.min(seg)
    hi = jnp.max(seg)
    hb = h.astype(jnp.bfloat16)
    iota_w = lax.broadcasted_iota(jnp.int32, (W, B), 0)
    for k in range(NW):
        @pl.when((hi >= k * W) & (lo < (k + 1) * W))
        def _window(k=k):
            local = seg - (k * W)
            ohf = (iota_w == local).astype(jnp.float32)
            pd = lax.dot_general(ohf.astype(jnp.bfloat16), hb,
                                 (((1,), (0,)), ((), ())),
                                 preferred_element_type=jnp.float32)
            accp_ref[k * W:(k + 1) * W, :] += pd
            cnt = jnp.sum(ohf, axis=1, keepdims=True)
            accc_ref[k * W:(k + 1) * W, :] += jnp.broadcast_to(cnt, (W, F))

    @pl.when(i == NT - 1)
    def _epilogue():
        inv_n = 1.0 / N
        mean = jnp.sum(accp_ref[...], axis=0, keepdims=True) * inv_n

        var = jnp.maximum(q_ref[...] * inv_n - mean * mean, 0.0)
        scale = g_ref[...] * lax.rsqrt(var + EPS)
        shift = be_ref[...] - mean * scale
        out_ref[...] = accp_ref[:G, :] * scale + accc_ref[:G, :] * shift


@jax.jit
def _fused(x, seg3, W1, b1, W2, b2, gamma, beta):
    return pl.pallas_call(
        _fused_body,
        grid=(NT,),
        in_specs=[
            pl.BlockSpec((B, F), lambda i: (i, 0)),
            pl.BlockSpec((1, 1, B), lambda i: (i, 0, 0)),
            pl.BlockSpec((F, F), lambda i: (0, 0)),
            pl.BlockSpec((1, F), lambda i: (0, 0)),
            pl.BlockSpec((F, F), lambda i: (0, 0)),
            pl.BlockSpec((1, F), lambda i: (0, 0)),
            pl.BlockSpec((1, F), lambda i: (0, 0)),
            pl.BlockSpec((1, F), lambda i: (0, 0)),
        ],
        out_specs=pl.BlockSpec((G, F), lambda i: (0, 0)),
        out_shape=jax.ShapeDtypeStruct((G, F), jnp.float32),
        scratch_shapes=[
            pltpu.VMEM((GP, F), jnp.float32),
            pltpu.VMEM((GP, F), jnp.float32),
            pltpu.VMEM((1, F), jnp.float32),
        ],
        compiler_params=pltpu.CompilerParams(
            dimension_semantics=("arbitrary",),
        ),
    )(x, seg3, W1, b1, W2, b2, gamma, beta)


def kernel(input, graph_indices, node_counts, W1, b1, W2, b2, gamma, beta):
    del node_counts
    seg3 = graph_indices.astype(jnp.int32).reshape(NT, 1, B)
    return _fused(input, seg3, W1, b1.reshape(1, F), W2, b2.reshape(1, F),
                  gamma.reshape(1, F), beta.reshape(1, F))


# speedup vs baseline: 1.2226x; 1.2226x over previous
import functools

import jax
import jax.numpy as jnp
from jax import lax
from jax.experimental import pallas as pl
from jax.experimental.pallas import tpu as pltpu

N = 100000
F = 128
G = 1000
B = 4000
NT = N // B
W = 64
NW = (G + W - 1) // W
GP = NW * W
EPS = 1e-5


def _fused_body(x_ref, seg_ref, w1_ref, b1_ref, w2_ref, b2_ref, g_ref,
                be_ref, out_ref, accp_ref, accc_ref, q_ref):
    i = pl.program_id(0)

    @pl.when(i == 0)
    def _init():
        accp_ref[...] = jnp.zeros_like(accp_ref)
        accc_ref[...] = jnp.zeros_like(accc_ref)
        q_ref[...] = jnp.zeros_like(q_ref)

    x = x_ref[...]
    l1 = lax.dot_general(x, w1_ref[...], (((1,), (1,)), ((), ())),
                         preferred_element_type=jnp.float32) + b1_ref[...]
    l2 = lax.dot_general(x, w2_ref[...], (((1,), (1,)), ((), ())),
                         preferred_element_type=jnp.float32) + b2_ref[...]
    h = l1 * l2

    q_ref[...] += jnp.sum(h * h, axis=0, keepdims=True)

    seg = seg_ref[0]
    lo = jnp.min(seg)
    hi = jnp.max(seg)
    hb = h.astype(jnp.bfloat16)
    iota_w = lax.broadcasted_iota(jnp.int32, (W, B), 0)
    for k in range(NW):
        @pl.when((hi >= k * W) & (lo < (k + 1) * W))
        def _window(k=k):
            local = seg - (k * W)
            ohf = (iota_w == local).astype(jnp.float32)
            pd = lax.dot_general(ohf.astype(jnp.bfloat16), hb,
                                 (((1,), (0,)), ((), ())),
                                 preferred_element_type=jnp.float32)
            accp_ref[k * W:(k + 1) * W, :] += pd
            cnt = jnp.sum(ohf, axis=1, keepdims=True)
            accc_ref[k * W:(k + 1) * W, :] += jnp.broadcast_to(cnt, (W, F))

    @pl.when(i == NT - 1)
    def _epilogue():
        inv_n = 1.0 / N
        mean = jnp.sum(accp_ref[...], axis=0, keepdims=True) * inv_n

        var = jnp.maximum(q_ref[...] * inv_n - mean * mean, 0.0)
        scale = g_ref[...] * lax.rsqrt(var + EPS)
        shift = be_ref[...] - mean * scale
        out_ref[...] = accp_ref[:G, :] * scale + accc_ref[:G, :] * shift


@jax.jit
def _fused(x, seg3, W1, b1, W2, b2, gamma, beta):
    return pl.pallas_call(
        _fused_body,
        grid=(NT,),
        in_specs=[
            pl.BlockSpec((B, F), lambda i: (i, 0)),
            pl.BlockSpec((1, 1, B), lambda i: (i, 0, 0)),
            pl.BlockSpec((F, F), lambda i: (0, 0)),
            pl.BlockSpec((1, F), lambda i: (0, 0)),
            pl.BlockSpec((F, F), lambda i: (0, 0)),
            pl.BlockSpec((1, F), lambda i: (0, 0)),
            pl.BlockSpec((1, F), lambda i: (0, 0)),
            pl.BlockSpec((1, F), lambda i: (0, 0)),
        ],
        out_specs=pl.BlockSpec((G, F), lambda i: (0, 0)),
        out_shape=jax.ShapeDtypeStruct((G, F), jnp.float32),
        scratch_shapes=[
            pltpu.VMEM((GP, F), jnp.float32),
            pltpu.VMEM((GP, F), jnp.float32),
            pltpu.VMEM((1, F), jnp.float32),
        ],
        compiler_params=pltpu.CompilerParams(
            dimension_semantics=("arbitrary",),
        ),
    )(x, seg3, W1, b1, W2, b2, gamma, beta)


def kernel(input, graph_indices, node_counts, W1, b1, W2, b2, gamma, beta):
    del node_counts
    seg3 = graph_indices.astype(jnp.int32).reshape(NT, 1, B)
    return _fused(input, seg3, W1, b1.reshape(1, F), W2, b2.reshape(1, F),
                  gamma.reshape(1, F), beta.reshape(1, F))
